# speedup vs baseline: 1.0729x; 1.0729x over previous
import functools

import jax
import jax.numpy as jnp
from jax import lax
from jax.experimental import pallas as pl
from jax.experimental.pallas import tpu as pltpu
from jax.experimental.pallas import tpu_sc as plsc

VOCAB = 400002
EMBED_DIM = 128
UNK_IDX = 0
PAD_IDX = VOCAB - 1

NC = 2
NS = 16
LANES = 16
NW = NC * NS

B_TOTAL = 1024 * 200
B_PER_W = B_TOTAL // NW
CHUNK = 128
NCHUNK = B_PER_W // CHUNK

_mesh = plsc.VectorSubcoreMesh(core_axis_name="c", subcore_axis_name="s")


@functools.partial(
    pl.kernel,
    out_type=jax.ShapeDtypeStruct((B_TOTAL, EMBED_DIM), jnp.float32),
    mesh=_mesh,
    scratch_types=[
        pltpu.VMEM((CHUNK,), jnp.int32),
        pltpu.VMEM((CHUNK, EMBED_DIM), jnp.float32),
        pltpu.SemaphoreType.DMA,
    ],
)
def _gather_kernel(idx_hbm, table_hbm, out_hbm, idx_v, rows_v, sem):
    wid = lax.axis_index("s") * NC + lax.axis_index("c")
    base0 = wid * B_PER_W

    def chunk_body(ci, carry):
        base = base0 + ci * CHUNK
        pltpu.sync_copy(idx_hbm.at[pl.ds(base, CHUNK)], idx_v)

        def clamp_body(vi, c):
            v = idx_v[pl.ds(vi * LANES, LANES)]
            v = jnp.where(v > PAD_IDX, UNK_IDX, v)
            v = jnp.where(v < 0, PAD_IDX, v)
            idx_v[pl.ds(vi * LANES, LANES)] = v
            return c

        lax.fori_loop(0, CHUNK // LANES, clamp_body, 0)

        pltpu.async_copy(table_hbm.at[idx_v], rows_v, sem).wait()
        pltpu.sync_copy(rows_v, out_hbm.at[pl.ds(base, CHUNK)])
        return carry

    lax.fori_loop(0, NCHUNK, chunk_body, 0)


def kernel(idxes, fixed_table):
    flat = idxes.reshape(-1)
    out = _gather_kernel(flat, fixed_table)
    return out.reshape(idxes.shape + (EMBED_DIM,))


# speedup vs baseline: 1.7535x; 1.6343x over previous
import functools

import jax
import jax.numpy as jnp
from jax import lax
from jax.experimental import pallas as pl
from jax.experimental.pallas import tpu as pltpu
from jax.experimental.pallas import tpu_sc as plsc

VOCAB = 400002
EMBED_DIM = 128
UNK_IDX = 0
PAD_IDX = VOCAB - 1

NC = 2
NS = 16
LANES = 16
NW = NC * NS

B_TOTAL = 1024 * 200
B_PER_W = B_TOTAL // NW
CHUNK = 128
NCHUNK = B_PER_W // CHUNK
NBUF = 5
NGROUP = NCHUNK // NBUF

_mesh = plsc.VectorSubcoreMesh(core_axis_name="c", subcore_axis_name="s")


@functools.partial(
    pl.kernel,
    out_type=jax.ShapeDtypeStruct((B_TOTAL, EMBED_DIM), jnp.float32),
    mesh=_mesh,
    scratch_types=[
        pltpu.VMEM((B_PER_W,), jnp.int32),
        [pltpu.VMEM((CHUNK, EMBED_DIM), jnp.float32) for _ in range(NBUF)],
        [pltpu.SemaphoreType.DMA for _ in range(NBUF)],
        [pltpu.SemaphoreType.DMA for _ in range(NBUF)],
    ],
)
def _gather_kernel(idx_hbm, table_hbm, out_hbm, idx_all, rows, gsem, wsem):
    wid = lax.axis_index("s") * NC + lax.axis_index("c")
    base0 = wid * B_PER_W

    pltpu.sync_copy(idx_hbm.at[pl.ds(base0, B_PER_W)], idx_all)

    def clamp_body(vi, c):
        v = idx_all[pl.ds(vi * LANES, LANES)]
        v = jnp.where(v > PAD_IDX, UNK_IDX, v)
        v = jnp.where(v < 0, PAD_IDX, v)
        idx_all[pl.ds(vi * LANES, LANES)] = v
        return c

    lax.fori_loop(0, B_PER_W // LANES, clamp_body, 0)

    def start_gather(ci, b):
        pltpu.async_copy(
            table_hbm.at[idx_all.at[pl.ds(ci * CHUNK, CHUNK)]], rows[b], gsem[b]
        )

    def wait_gather(b):
        pltpu.make_async_copy(
            table_hbm.at[idx_all.at[pl.ds(0, CHUNK)]], rows[b], gsem[b]
        ).wait()

    def start_writeback(ci, b):
        pltpu.async_copy(
            rows[b], out_hbm.at[pl.ds(base0 + ci * CHUNK, CHUNK)], wsem[b]
        )

    def wait_writeback(b):
        pltpu.make_async_copy(
            rows[b], out_hbm.at[pl.ds(base0, CHUNK)], wsem[b]
        ).wait()

    for b in range(NBUF):
        start_gather(b, b)

    def group_body(g, c):
        for b in range(NBUF):
            ci = g * NBUF + b
            wait_gather(b)
            start_writeback(ci, b)
            wait_writeback(b)
            start_gather(ci + NBUF, b)
        return c

    lax.fori_loop(0, NGROUP - 1, group_body, 0)

    for b in range(NBUF):
        ci = (NGROUP - 1) * NBUF + b
        wait_gather(b)
        start_writeback(ci, b)
    for b in range(NBUF):
        wait_writeback(b)


def kernel(idxes, fixed_table):
    flat = idxes.reshape(-1)
    out = _gather_kernel(flat, fixed_table)
    return out.reshape(idxes.shape + (EMBED_DIM,))
